# baseline (speedup 1.0000x reference)
import jax
import jax.numpy as jnp
from jax import lax
from jax.experimental import pallas as pl
from jax.experimental.pallas import tpu as pltpu
from jax.experimental.pallas import tpu_sc as plsc

T_OUT = 8192
T_IN = 4096
L = 16
NC = 2
NS = 16
NW = NC * NS
SC_ROWS = 2048
TC_ROWS = T_OUT - SC_ROWS
R = SC_ROWS // NW
CH = 8
U = 8
NCHUNK = R // CH
SLICE = T_IN // NS
BR = 1536


def _sc_body(att_hbm, out_hbm, buf, hist, tmp, acc, shared, sems, csem):
    cid = lax.axis_index("c")
    sid = lax.axis_index("s")
    wid = sid * NC + cid
    base = TC_ROWS + wid * R

    lane = lax.iota(jnp.int32, L)
    zeros_i = jnp.zeros((L,), jnp.int32)
    ones_i = jnp.ones((L,), jnp.int32)

    def _zero(i, _):
        hist[pl.ds(i * L, L)] = zeros_i
        return 0
    lax.fori_loop(0, T_IN // L, _zero, 0)

    def process(bufb):
        def row_body(r, _):
            def step(i, carry):
                best, beststep = carry
                j0 = i * U
                for u in range(U):
                    v = bufb[r, pl.ds((j0 + u) * L, L)]
                    m = v > best
                    best = jnp.maximum(best, v)
                    beststep = jnp.where(
                        m, jnp.full((L,), j0 + u, jnp.int32), beststep
                    )
                return best, beststep

            best0 = jnp.full((L,), -jnp.inf, jnp.float32)
            best, beststep = lax.fori_loop(
                0, T_IN // L // U, step, (best0, zeros_i)
            )
            col = beststep * L + lane
            m, mi = best, col
            for shift in (8, 4, 2, 1):
                perm = (lane + shift) & (L - 1)
                m2 = m.at[perm].get(mode="promise_in_bounds")
                i2 = mi.at[perm].get(mode="promise_in_bounds")
                take = (m2 > m) | ((m2 == m) & (i2 < mi))
                m = jnp.where(take, m2, m)
                mi = jnp.where(take, i2, mi)
            plsc.addupdate_scatter(hist, [mi], ones_i, mask=lane == 0)
            return 0
        lax.fori_loop(0, CH, row_body, 0)

    def issue(ch, b):
        return pltpu.make_async_copy(
            att_hbm.at[pl.ds(base + ch * CH, CH)], buf.at[b], sems.at[b]
        )

    issue(0, 0).start()

    def pair_body(p, _):
        ch0 = p * 2
        issue(ch0 + 1, 1).start()
        issue(ch0, 0).wait()
        process(buf.at[0])

        @pl.when(p < NCHUNK // 2 - 1)
        def _():
            issue(ch0 + 2, 0).start()
        issue(ch0 + 1, 1).wait()
        process(buf.at[1])
        return 0
    lax.fori_loop(0, NCHUNK // 2, pair_body, 0)

    pltpu.sync_copy(hist, shared.at[sid])
    plsc.subcore_barrier()

    colbase = sid * SLICE
    descs = [
        pltpu.make_async_copy(
            shared.at[t, pl.ds(colbase, SLICE)], tmp.at[t], csem
        )
        for t in range(NS)
    ]
    for d in descs:
        d.start()
    for d in descs:
        d.wait()

    for i in range(SLICE // L):
        s = pl.ds(i * L, L)
        v = tmp[0, s]
        for t in range(1, NS):
            v = v + tmp[t, s]
        acc[s] = v

    pltpu.sync_copy(acc, out_hbm.at[cid, pl.ds(colbase, SLICE)])


def _tc_hist_body(x_ref, o_ref):
    i = pl.program_id(0)
    x = x_ref[...]
    cols = lax.broadcasted_iota(jnp.int32, (BR, T_IN), 1)
    rowmax = jnp.max(x, axis=-1, keepdims=True)
    am = jnp.min(jnp.where(x == rowmax, cols, T_IN), axis=-1)
    h = jnp.sum((am[:, None] == cols).astype(jnp.int32), axis=0)

    @pl.when(i == 0)
    def _():
        o_ref[...] = h

    @pl.when(i > 0)
    def _():
        o_ref[...] = o_ref[...] + h


def _merge_body(p_ref, t_ref, o_ref):
    o_ref[...] = p_ref[0] + p_ref[1] + t_ref[...]


@jax.jit
def kernel(att_ws):
    tc_partial = pl.pallas_call(
        _tc_hist_body,
        grid=(TC_ROWS // BR,),
        in_specs=[pl.BlockSpec((BR, T_IN), lambda i: (i, 0))],
        out_specs=pl.BlockSpec((T_IN,), lambda i: (0,)),
        out_shape=jax.ShapeDtypeStruct((T_IN,), jnp.int32),
    )(att_ws)
    mesh = plsc.VectorSubcoreMesh(
        core_axis_name="c", subcore_axis_name="s", num_cores=NC, num_subcores=NS
    )
    partials = pl.kernel(
        _sc_body,
        out_type=jax.ShapeDtypeStruct((NC, T_IN), jnp.int32),
        mesh=mesh,
        compiler_params=pltpu.CompilerParams(needs_layout_passes=False),
        scratch_types=[
            pltpu.VMEM((2, CH, T_IN), jnp.float32),
            pltpu.VMEM((T_IN,), jnp.int32),
            pltpu.VMEM((NS, SLICE), jnp.int32),
            pltpu.VMEM((SLICE,), jnp.int32),
            pltpu.VMEM_SHARED((NS, T_IN), jnp.int32),
            pltpu.SemaphoreType.DMA((2,)),
            pltpu.SemaphoreType.DMA,
        ],
    )(att_ws)
    out = pl.pallas_call(
        _merge_body,
        out_shape=jax.ShapeDtypeStruct((T_IN,), jnp.int32),
    )(partials, tc_partial)
    return out


# speedup vs baseline: 1.0322x; 1.0322x over previous
import jax
import jax.numpy as jnp
from jax import lax
from jax.experimental import pallas as pl
from jax.experimental.pallas import tpu as pltpu
from jax.experimental.pallas import tpu_sc as plsc

T_OUT = 8192
T_IN = 4096
L = 16
NC = 2
NS = 16
NW = NC * NS
SC_ROWS = 3072
TC_ROWS = T_OUT - SC_ROWS
R = SC_ROWS // NW
CH = 8
U = 8
NCHUNK = R // CH
SLICE = T_IN // NS
BR = 1024


def _sc_body(att_hbm, out_hbm, buf, hist, tmp, acc, shared, sems, csem):
    cid = lax.axis_index("c")
    sid = lax.axis_index("s")
    wid = sid * NC + cid
    base = TC_ROWS + wid * R

    lane = lax.iota(jnp.int32, L)
    zeros_i = jnp.zeros((L,), jnp.int32)
    ones_i = jnp.ones((L,), jnp.int32)

    def _zero(i, _):
        hist[pl.ds(i * L, L)] = zeros_i
        return 0
    lax.fori_loop(0, T_IN // L, _zero, 0)

    def process(bufb):
        def row_body(r, _):
            def step(i, carry):
                best, beststep = carry
                j0 = i * U
                for u in range(U):
                    v = bufb[r, pl.ds((j0 + u) * L, L)]
                    m = v > best
                    best = jnp.maximum(best, v)
                    beststep = jnp.where(
                        m, jnp.full((L,), j0 + u, jnp.int32), beststep
                    )
                return best, beststep

            best0 = jnp.full((L,), -jnp.inf, jnp.float32)
            best, beststep = lax.fori_loop(
                0, T_IN // L // U, step, (best0, zeros_i)
            )
            col = beststep * L + lane
            m, mi = best, col
            for shift in (8, 4, 2, 1):
                perm = (lane + shift) & (L - 1)
                m2 = m.at[perm].get(mode="promise_in_bounds")
                i2 = mi.at[perm].get(mode="promise_in_bounds")
                take = (m2 > m) | ((m2 == m) & (i2 < mi))
                m = jnp.where(take, m2, m)
                mi = jnp.where(take, i2, mi)
            plsc.addupdate_scatter(hist, [mi], ones_i, mask=lane == 0)
            return 0
        lax.fori_loop(0, CH, row_body, 0)

    def issue(ch, b):
        return pltpu.make_async_copy(
            att_hbm.at[pl.ds(base + ch * CH, CH)], buf.at[b], sems.at[b]
        )

    issue(0, 0).start()

    def pair_body(p, _):
        ch0 = p * 2
        issue(ch0 + 1, 1).start()
        issue(ch0, 0).wait()
        process(buf.at[0])

        @pl.when(p < NCHUNK // 2 - 1)
        def _():
            issue(ch0 + 2, 0).start()
        issue(ch0 + 1, 1).wait()
        process(buf.at[1])
        return 0
    lax.fori_loop(0, NCHUNK // 2, pair_body, 0)

    pltpu.sync_copy(hist, shared.at[sid])
    plsc.subcore_barrier()

    colbase = sid * SLICE
    descs = [
        pltpu.make_async_copy(
            shared.at[t, pl.ds(colbase, SLICE)], tmp.at[t], csem
        )
        for t in range(NS)
    ]
    for d in descs:
        d.start()
    for d in descs:
        d.wait()

    for i in range(SLICE // L):
        s = pl.ds(i * L, L)
        v = tmp[0, s]
        for t in range(1, NS):
            v = v + tmp[t, s]
        acc[s] = v

    pltpu.sync_copy(acc, out_hbm.at[cid, pl.ds(colbase, SLICE)])


def _tc_hist_body(x_ref, o_ref):
    i = pl.program_id(0)
    x = x_ref[...]
    cols = lax.broadcasted_iota(jnp.int32, (BR, T_IN), 1)
    rowmax = jnp.max(x, axis=-1, keepdims=True)
    am = jnp.min(jnp.where(x == rowmax, cols, T_IN), axis=-1)
    h = jnp.sum((am[:, None] == cols).astype(jnp.int32), axis=0)

    @pl.when(i == 0)
    def _():
        o_ref[...] = h

    @pl.when(i > 0)
    def _():
        o_ref[...] = o_ref[...] + h


def _merge_body(p_ref, t_ref, o_ref):
    o_ref[...] = p_ref[0] + p_ref[1] + t_ref[...]


@jax.jit
def kernel(att_ws):
    tc_partial = pl.pallas_call(
        _tc_hist_body,
        grid=(TC_ROWS // BR,),
        in_specs=[pl.BlockSpec((BR, T_IN), lambda i: (i, 0))],
        out_specs=pl.BlockSpec((T_IN,), lambda i: (0,)),
        out_shape=jax.ShapeDtypeStruct((T_IN,), jnp.int32),
    )(att_ws)
    mesh = plsc.VectorSubcoreMesh(
        core_axis_name="c", subcore_axis_name="s", num_cores=NC, num_subcores=NS
    )
    partials = pl.kernel(
        _sc_body,
        out_type=jax.ShapeDtypeStruct((NC, T_IN), jnp.int32),
        mesh=mesh,
        compiler_params=pltpu.CompilerParams(needs_layout_passes=False),
        scratch_types=[
            pltpu.VMEM((2, CH, T_IN), jnp.float32),
            pltpu.VMEM((T_IN,), jnp.int32),
            pltpu.VMEM((NS, SLICE), jnp.int32),
            pltpu.VMEM((SLICE,), jnp.int32),
            pltpu.VMEM_SHARED((NS, T_IN), jnp.int32),
            pltpu.SemaphoreType.DMA((2,)),
            pltpu.SemaphoreType.DMA,
        ],
    )(att_ws)
    out = pl.pallas_call(
        _merge_body,
        out_shape=jax.ShapeDtypeStruct((T_IN,), jnp.int32),
    )(partials, tc_partial)
    return out
